# baseline (speedup 1.0000x reference)
import functools
import math

import jax
import jax.numpy as jnp
from jax import lax
from jax.experimental import pallas as pl
from jax.experimental.pallas import tpu as pltpu
from jax.experimental.pallas import tpu_sc as plsc

D_MODEL = 64
D_PAD = 128
SCALE = math.sqrt(D_MODEL)

_info = plsc.get_sparse_core_info()
NC, NS, L = _info.num_cores, _info.num_subcores, _info.num_lanes
NW = NC * NS

GCHUNK = 40
NBUF = 4


def _emb_body(rows_per_w, seq, lut_hbm, x_hbm, out_hbm, idx_v, *bufs_and_sems):
    rbuf = bufs_and_sems[:NBUF]
    isem = bufs_and_sems[NBUF]
    gsem = bufs_and_sems[NBUF + 1:NBUF + 1 + NBUF]
    ssem = bufs_and_sems[NBUF + 1 + NBUF:]

    halves = seq // GCHUNK

    wid = lax.axis_index("s") * NC + lax.axis_index("c")
    row0 = wid * rows_per_w

    pltpu.async_copy(x_hbm.at[pl.ds(row0, rows_per_w)], idx_v, isem).wait()

    def start_gathers(b, r):
        for h in range(halves):
            pltpu.async_copy(
                lut_hbm.at[idx_v.at[r, pl.ds(h * GCHUNK, GCHUNK)]],
                rbuf[b].at[pl.ds(h * GCHUNK, GCHUNK)],
                gsem[b],
            )

    def wait_gathers(b, r):
        for h in range(halves):
            pltpu.make_async_copy(
                lut_hbm.at[idx_v.at[r, pl.ds(h * GCHUNK, GCHUNK)]],
                rbuf[b].at[pl.ds(h * GCHUNK, GCHUNK)],
                gsem[b],
            ).wait()

    def start_scatter(b, r):
        pltpu.async_copy(
            rbuf[b], out_hbm.at[row0 + r, :, pl.ds(0, D_MODEL)], ssem[b])

    def wait_scatter(b, r):
        pltpu.make_async_copy(
            rbuf[b], out_hbm.at[row0 + r, :, pl.ds(0, D_MODEL)], ssem[b]).wait()

    def scale(b):
        buf = rbuf[b]

        @plsc.parallel_loop(0, seq, 1, unroll=8)
        def _(r):
            for t in range(D_MODEL // L):
                sl = pl.ds(t * L, L)
                buf[r, sl] = buf[r, sl] * SCALE

    n_groups = rows_per_w // NBUF

    for b in range(NBUF - 1):
        start_gathers(b, b)

    for b in range(NBUF):
        wait_gathers(b, b)
        scale(b)
        start_scatter(b, b)
        bn = (b + NBUF - 1) % NBUF
        if b == 0:
            start_gathers(NBUF - 1, NBUF - 1)
        else:
            wait_scatter(bn, b - 1)
            start_gathers(bn, b + NBUF - 1)

    def group_body(g, _):
        for b in range(NBUF):
            r = g * NBUF + b
            wait_gathers(b, r)
            scale(b)
            start_scatter(b, r)
            bn = (b + NBUF - 1) % NBUF
            wait_scatter(bn, r - 1)
            start_gathers(bn, r + NBUF - 1)
        return 0

    lax.fori_loop(1, n_groups - 1, group_body, 0)

    for b in range(NBUF):
        r = (n_groups - 1) * NBUF + b
        wait_gathers(b, r)
        scale(b)
        start_scatter(b, r)
        if r + NBUF - 1 < rows_per_w:
            bn = (b + NBUF - 1) % NBUF
            wait_scatter(bn, r - 1)
            start_gathers(bn, r + NBUF - 1)

    for r in range(rows_per_w - NBUF, rows_per_w):
        wait_scatter(r % NBUF, r)


def kernel(x, lut):
    B, S = x.shape
    V, D = lut.shape
    assert B % NW == 0 and S % GCHUNK == 0 and D == D_MODEL
    rows_per_w = B // NW

    lut_view = jnp.pad(lut, ((0, 0), (0, D_PAD - D))).reshape(2 * V, D)
    x2 = x * 2

    scratch = [pltpu.VMEM((rows_per_w, S), jnp.int32)]
    scratch += [pltpu.VMEM((S, D), jnp.float32) for _ in range(NBUF)]
    scratch += [pltpu.SemaphoreType.DMA for _ in range(1 + 2 * NBUF)]

    mesh = plsc.VectorSubcoreMesh(core_axis_name="c", subcore_axis_name="s")
    k = functools.partial(
        pl.kernel,
        mesh=mesh,
        out_type=jax.ShapeDtypeStruct((B, S, D_PAD), jnp.float32),
        scratch_types=scratch,
        compiler_params=pltpu.CompilerParams(use_tc_tiling_on_sc=False),
    )(functools.partial(_emb_body, rows_per_w, S))

    out = k(lut_view, x2)
    return out[..., :D_MODEL]
